# baseline (speedup 1.0000x reference)
import functools

import jax
import jax.numpy as jnp
from jax import lax
from jax.experimental import pallas as pl
from jax.experimental.pallas import tpu as pltpu
from jax.experimental.pallas import tpu_sc as plsc

_DIM = 16
_BATCH = 16384
_N_SCORE = 32
_NUM_ITEMS = 100000

_NC, _NS = 2, 16
_NW = _NC * _NS
_B_W = _BATCH // _NW
_CHUNK = 128
_N_CHUNK = _B_W // _CHUNK

_GRID = 8
_BB = _BATCH // _GRID
_BI = 12800


def _sc_gather(user_emb, item_emb, user_ids, pos_ids, neg_ids, score_ids):
    mesh = plsc.VectorSubcoreMesh(core_axis_name="c", subcore_axis_name="s")

    @functools.partial(
        pl.kernel,
        mesh=mesh,
        compiler_params=pltpu.CompilerParams(use_tc_tiling_on_sc=False),
        out_type=[
            jax.ShapeDtypeStruct((_BATCH, _DIM), jnp.float32),
            jax.ShapeDtypeStruct((_BATCH, _DIM), jnp.float32),
            jax.ShapeDtypeStruct((_BATCH, _DIM), jnp.float32),
            jax.ShapeDtypeStruct((_N_SCORE, _DIM), jnp.float32),
        ],
        scratch_types=[
            pltpu.VMEM((_N_CHUNK, _CHUNK), jnp.int32),
            pltpu.VMEM((_N_CHUNK, _CHUNK), jnp.int32),
            pltpu.VMEM((_N_CHUNK, _CHUNK), jnp.int32),
            pltpu.VMEM((_B_W, _DIM), jnp.float32),
            pltpu.VMEM((_B_W, _DIM), jnp.float32),
            pltpu.VMEM((_B_W, _DIM), jnp.float32),
            pltpu.VMEM((_N_SCORE,), jnp.int32),
            pltpu.VMEM((_N_SCORE, _DIM), jnp.float32),
            pltpu.SemaphoreType.DMA,
        ],
    )
    def k(user_hbm, item_hbm, uid_hbm, pid_hbm, nid_hbm, sid_hbm,
          u_hbm, p_hbm, n_hbm, su_hbm,
          uid_v, pid_v, nid_v, u_v, p_v, n_v, sid_v, su_v, sem):
        wid = lax.axis_index("s") * _NC + lax.axis_index("c")
        base = wid * _B_W

        for c in range(_N_CHUNK):
            off = base + c * _CHUNK
            pltpu.sync_copy(uid_hbm.at[pl.ds(off, _CHUNK)], uid_v.at[c])
            pltpu.sync_copy(pid_hbm.at[pl.ds(off, _CHUNK)], pid_v.at[c])
            pltpu.sync_copy(nid_hbm.at[pl.ds(off, _CHUNK)], nid_v.at[c])

        copies = []
        for c in range(_N_CHUNK):
            dst = pl.ds(c * _CHUNK, _CHUNK)
            copies.append(pltpu.async_copy(user_hbm.at[uid_v.at[c]], u_v.at[dst], sem))
            copies.append(pltpu.async_copy(item_hbm.at[pid_v.at[c]], p_v.at[dst], sem))
            copies.append(pltpu.async_copy(item_hbm.at[nid_v.at[c]], n_v.at[dst], sem))
        for cp in copies:
            cp.wait()

        dst = pl.ds(base, _B_W)
        pltpu.sync_copy(u_v, u_hbm.at[dst])
        pltpu.sync_copy(p_v, p_hbm.at[dst])
        pltpu.sync_copy(n_v, n_hbm.at[dst])

        @pl.when(wid == 0)
        def _():
            pltpu.sync_copy(sid_hbm, sid_v)
            pltpu.async_copy(user_hbm.at[sid_v], su_v, sem).wait()
            pltpu.sync_copy(su_v, su_hbm)

    return k(user_emb, item_emb, user_ids, pos_ids, neg_ids, score_ids)


def _tc_compute(u_rows, p_rows, n_rows, su_vec, item_emb):
    def body(u_ref, p_ref, n_ref, su_ref, it_ref, pos_ref, neg_ref, out_ref):
        u = u_ref[...]
        dp = u - p_ref[...]
        dn = u - n_ref[...]
        pos_ref[...] = jnp.sum(dp * dp, axis=1)
        neg_ref[...] = jnp.sum(dn * dn, axis=1)

        su = su_ref[...]
        it = it_ref[...]
        dots = lax.dot_general(su, it, (((1,), (1,)), ((), ())),
                               preferred_element_type=jnp.float32)
        su2 = jnp.sum(su * su, axis=1)
        it2 = jnp.sum(it * it, axis=1)
        out_ref[...] = 2.0 * dots - su2[:, None] - it2[None, :]

    return pl.pallas_call(
        body,
        grid=(_GRID,),
        in_specs=[
            pl.BlockSpec((_BB, _DIM), lambda i: (i, 0)),
            pl.BlockSpec((_BB, _DIM), lambda i: (i, 0)),
            pl.BlockSpec((_BB, _DIM), lambda i: (i, 0)),
            pl.BlockSpec((_N_SCORE, _DIM), lambda i: (0, 0)),
            pl.BlockSpec((_BI, _DIM), lambda i: (i, 0)),
        ],
        out_specs=[
            pl.BlockSpec((_BB,), lambda i: (i,)),
            pl.BlockSpec((_BB,), lambda i: (i,)),
            pl.BlockSpec((_N_SCORE, _BI), lambda i: (0, i)),
        ],
        out_shape=[
            jax.ShapeDtypeStruct((_BATCH,), jnp.float32),
            jax.ShapeDtypeStruct((_BATCH,), jnp.float32),
            jax.ShapeDtypeStruct((_N_SCORE, _NUM_ITEMS), jnp.float32),
        ],
    )(u_rows, p_rows, n_rows, su_vec, item_emb)


def kernel(user_embeddings, item_embeddings, user_ids, pos_item_ids,
           neg_item_ids, score_user_ids):
    u_rows, p_rows, n_rows, su_vec = _sc_gather(
        user_embeddings, item_embeddings, user_ids, pos_item_ids,
        neg_item_ids, score_user_ids)
    pos_d, neg_d, scores = _tc_compute(
        u_rows, p_rows, n_rows, su_vec, item_embeddings)
    return (pos_d, neg_d, scores)
